# baseline (speedup 1.0000x reference)
import jax
import jax.numpy as jnp
from jax import lax
from jax.experimental import pallas as pl
from jax.experimental.pallas import tpu as pltpu
from jax.experimental.pallas import tpu_sc as plsc

_NUM_CORES = 2
_NUM_SUBCORES = 16
_NUM_WORKERS = _NUM_CORES * _NUM_SUBCORES
_LANE_PAD = 128
_LANES = 16


def kernel(session_week_id, emb_weight):
    batch = session_week_id.shape[0]
    dim = emb_weight.shape[1]
    idx = session_week_id.astype(jnp.int32)
    idx_eo = idx.reshape(batch // 2, 2).T.reshape(2, batch // 2)
    table = jnp.pad(emb_weight, ((0, 0), (0, _LANE_PAD - dim)))

    half = batch // 2
    h_per_w = half // _NUM_WORKERS
    n_chunks = 4
    pairs = h_per_w // n_chunks

    mesh = plsc.VectorSubcoreMesh(core_axis_name="c", subcore_axis_name="s")

    @pl.kernel(
        out_type=jax.ShapeDtypeStruct((half, _LANE_PAD), emb_weight.dtype),
        mesh=mesh,
        scratch_types=[
            pltpu.VMEM((2, h_per_w), jnp.int32),
            pltpu.VMEM((pairs, _LANE_PAD), emb_weight.dtype),
            pltpu.VMEM((pairs, _LANE_PAD), emb_weight.dtype),
            pltpu.VMEM((pairs, _LANE_PAD), emb_weight.dtype),
            pltpu.VMEM((pairs, _LANE_PAD), emb_weight.dtype),
            pltpu.VMEM((pairs, _LANE_PAD), emb_weight.dtype),
            pltpu.VMEM((pairs, _LANE_PAD), emb_weight.dtype),
            pltpu.SemaphoreType.DMA,
            pltpu.SemaphoreType.DMA,
            pltpu.SemaphoreType.DMA,
            pltpu.SemaphoreType.DMA,
            pltpu.SemaphoreType.DMA,
            pltpu.SemaphoreType.DMA,
        ],
    )
    def _gather(
        table_hbm,
        idx_hbm,
        out_hbm,
        idx_v,
        ae0,
        ae1,
        ao0,
        ao1,
        b0,
        b1,
        ge0,
        ge1,
        go0,
        go1,
        w0,
        w1,
    ):
        wid = lax.axis_index("s") * _NUM_CORES + lax.axis_index("c")
        base = wid * h_per_w
        pltpu.sync_copy(idx_hbm.at[:, pl.ds(base, h_per_w)], idx_v)
        bufs_e = (ae0, ae1)
        bufs_o = (ao0, ao1)
        bufs_b = (b0, b1)
        gsems_e = (ge0, ge1)
        gsems_o = (go0, go1)
        wsems = (w0, w1)
        ge_handles = [None, None]
        go_handles = [None, None]
        write_handles = [None, None]

        def start_gathers(k, b):
            ge_handles[b] = pltpu.async_copy(
                table_hbm.at[idx_v.at[0, pl.ds(k * pairs, pairs)]],
                bufs_e[b],
                gsems_e[b],
            )
            go_handles[b] = pltpu.async_copy(
                table_hbm.at[idx_v.at[1, pl.ds(k * pairs, pairs)]],
                bufs_o[b],
                gsems_o[b],
            )

        start_gathers(0, 0)
        for k in range(n_chunks):
            b = k % 2
            ge_handles[b].wait()
            go_handles[b].wait()
            if k + 1 < n_chunks:
                start_gathers(k + 1, 1 - b)
            if write_handles[b] is not None:
                write_handles[b].wait()
            e_ref = bufs_e[b]
            o_ref = bufs_o[b]
            b_ref = bufs_b[b]

            @pl.loop(0, pairs)
            def _(j):
                for s in range(dim // _LANES):
                    b_ref[j, pl.ds(s * _LANES, _LANES)] = e_ref[
                        j, pl.ds(s * _LANES, _LANES)
                    ]
                for s in range(dim // _LANES):
                    b_ref[j, pl.ds(dim + s * _LANES, _LANES)] = o_ref[
                        j, pl.ds(s * _LANES, _LANES)
                    ]

            write_handles[b] = pltpu.async_copy(
                b_ref,
                out_hbm.at[pl.ds(base + k * pairs, pairs)],
                wsems[b],
            )
        write_handles[0].wait()
        write_handles[1].wait()

    return _gather(table, idx_eo).reshape(batch, dim)


# speedup vs baseline: 1.3731x; 1.3731x over previous
import jax
import jax.numpy as jnp
from jax import lax
from jax.experimental import pallas as pl
from jax.experimental.pallas import tpu as pltpu
from jax.experimental.pallas import tpu_sc as plsc

_NUM_CORES = 2
_NUM_SUBCORES = 16
_NUM_WORKERS = _NUM_CORES * _NUM_SUBCORES
_LANE_PAD = 128


def kernel(session_week_id, emb_weight):
    batch = session_week_id.shape[0]
    dim = emb_weight.shape[1]
    b_per_w = batch // _NUM_WORKERS
    idx = session_week_id.astype(jnp.int32)
    table = jnp.pad(emb_weight, ((0, 0), (0, _LANE_PAD - dim)))

    mesh = plsc.VectorSubcoreMesh(core_axis_name="c", subcore_axis_name="s")

    @pl.kernel(
        out_type=jax.ShapeDtypeStruct((batch, _LANE_PAD), emb_weight.dtype),
        mesh=mesh,
        scratch_types=[
            pltpu.VMEM((b_per_w,), jnp.int32),
            pltpu.VMEM((b_per_w, _LANE_PAD), emb_weight.dtype),
            pltpu.SemaphoreType.DMA,
        ],
    )
    def _gather(table_hbm, idx_hbm, out_hbm, idx_v, rows_v, sem):
        wid = lax.axis_index("s") * _NUM_CORES + lax.axis_index("c")
        base = wid * b_per_w
        pltpu.sync_copy(idx_hbm.at[pl.ds(base, b_per_w)], idx_v)
        pltpu.async_copy(table_hbm.at[idx_v], rows_v, sem).wait()
        pltpu.sync_copy(rows_v, out_hbm.at[pl.ds(base, b_per_w)])

    return _gather(table, idx)[:, :dim]
